# Initial kernel scaffold; baseline (speedup 1.0000x reference)
import jax
import jax.numpy as jnp
from jax.experimental import pallas as pl


def kernel(char_seq, mapping_weight, char_emb_weight):
    raise NotImplementedError("write your pallas kernel here")


# speedup vs baseline: 6.3817x; 6.3817x over previous
import functools

import jax
import jax.numpy as jnp
from jax import lax
from jax.experimental import pallas as pl
from jax.experimental.pallas import tpu as pltpu
from jax.experimental.pallas import tpu_sc as plsc

CHARSET = 1000
UNIVERSAL = 1024
DIM = 128
BATCH = 4096
SEQ = 200

NC = 2
NS = 16
NW = NC * NS

TOTAL = BATCH * SEQ
PER_W = TOTAL // NW
CHUNK = 128
NCHUNK = PER_W // CHUNK


def _matmul_body(a_ref, b_ref, o_ref):
    o_ref[...] = jnp.dot(a_ref[...], b_ref[...],
                         preferred_element_type=jnp.float32)


def _compute_table(mapping_weight, char_emb_weight):
    return pl.pallas_call(
        _matmul_body,
        out_shape=jax.ShapeDtypeStruct((CHARSET, DIM), jnp.float32),
    )(mapping_weight, char_emb_weight)


_mesh = plsc.VectorSubcoreMesh(core_axis_name="c", subcore_axis_name="s")


@functools.partial(
    pl.kernel,
    mesh=_mesh,
    out_type=jax.ShapeDtypeStruct((TOTAL, DIM), jnp.float32),
    scratch_types=[
        pltpu.VMEM((NCHUNK, CHUNK), jnp.int32),
        pltpu.VMEM((CHUNK, DIM), jnp.float32),
        pltpu.SemaphoreType.DMA,
    ],
)
def _sc_gather(table_hbm, idx_hbm, out_hbm, idx_v, rows_v, sem):
    wid = lax.axis_index("s") * NC + lax.axis_index("c")
    pltpu.sync_copy(idx_hbm.at[wid], idx_v)
    base0 = wid * PER_W

    def body(j, carry):
        pltpu.async_copy(table_hbm.at[idx_v.at[j]], rows_v, sem).wait()
        pltpu.sync_copy(rows_v, out_hbm.at[pl.ds(base0 + j * CHUNK, CHUNK)])
        return carry

    lax.fori_loop(0, NCHUNK, body, 0)


def kernel(char_seq, mapping_weight, char_emb_weight):
    table = _compute_table(mapping_weight, char_emb_weight)
    idx = char_seq.reshape(NW, NCHUNK, CHUNK).astype(jnp.int32)
    out = _sc_gather(table, idx)
    return out.reshape(BATCH, SEQ, DIM)


# speedup vs baseline: 7.2128x; 1.1302x over previous
import functools

import jax
import jax.numpy as jnp
from jax import lax
from jax.experimental import pallas as pl
from jax.experimental.pallas import tpu as pltpu
from jax.experimental.pallas import tpu_sc as plsc

CHARSET = 1000
UNIVERSAL = 1024
DIM = 128
BATCH = 4096
SEQ = 200

NC = 2
NS = 16
NW = NC * NS

TOTAL = BATCH * SEQ
PER_W = TOTAL // NW
CHUNK = 128
NCHUNK = PER_W // CHUNK


def _matmul_body(a_ref, b_ref, o_ref):
    o_ref[...] = jnp.dot(a_ref[...], b_ref[...],
                         preferred_element_type=jnp.float32)


def _compute_table(mapping_weight, char_emb_weight):
    return pl.pallas_call(
        _matmul_body,
        out_shape=jax.ShapeDtypeStruct((CHARSET, DIM), jnp.float32),
    )(mapping_weight, char_emb_weight)


_mesh = plsc.VectorSubcoreMesh(core_axis_name="c", subcore_axis_name="s")


@functools.partial(
    pl.kernel,
    mesh=_mesh,
    out_type=jax.ShapeDtypeStruct((TOTAL, DIM), jnp.float32),
    scratch_types=[
        pltpu.VMEM((NCHUNK, CHUNK), jnp.int32),
        pltpu.VMEM((2, CHUNK, DIM), jnp.float32),
        pltpu.SemaphoreType.DMA,
    ],
)
def _sc_gather(table_hbm, idx_hbm, out_hbm, idx_v, rows_v, sem):
    wid = lax.axis_index("s") * NC + lax.axis_index("c")
    pltpu.sync_copy(idx_hbm.at[wid], idx_v)
    base0 = wid * PER_W

    pltpu.async_copy(table_hbm.at[idx_v.at[0]], rows_v.at[0], sem)

    def body(j, carry):
        buf = lax.rem(j, 2)
        pltpu.make_async_copy(table_hbm.at[idx_v.at[j]], rows_v.at[buf],
                              sem).wait()

        @pl.when(j + 1 < NCHUNK)
        def _():
            pltpu.async_copy(table_hbm.at[idx_v.at[j + 1]],
                             rows_v.at[1 - buf], sem)

        pltpu.sync_copy(rows_v.at[buf],
                        out_hbm.at[pl.ds(base0 + j * CHUNK, CHUNK)])
        return carry

    lax.fori_loop(0, NCHUNK, body, 0)


def kernel(char_seq, mapping_weight, char_emb_weight):
    table = _compute_table(mapping_weight, char_emb_weight)
    idx = char_seq.reshape(NW, NCHUNK, CHUNK).astype(jnp.int32)
    out = _sc_gather(table, idx)
    return out.reshape(BATCH, SEQ, DIM)


# speedup vs baseline: 16.4351x; 2.2786x over previous
import functools

import jax
import jax.numpy as jnp
from jax import lax
from jax.experimental import pallas as pl
from jax.experimental.pallas import tpu as pltpu
from jax.experimental.pallas import tpu_sc as plsc

CHARSET = 1000
UNIVERSAL = 1024
DIM = 128
BATCH = 4096
SEQ = 200

NC = 2
NS = 16
NW = NC * NS

TOTAL = BATCH * SEQ
PER_W = TOTAL // NW
CHUNK = 128
NCHUNK = PER_W // CHUNK


def _matmul_body(a_ref, b_ref, o_ref):
    o_ref[...] = jnp.dot(a_ref[...], b_ref[...],
                         preferred_element_type=jnp.float32)


def _compute_table(mapping_weight, char_emb_weight):
    return pl.pallas_call(
        _matmul_body,
        out_shape=jax.ShapeDtypeStruct((CHARSET, DIM), jnp.float32),
    )(mapping_weight, char_emb_weight)


_mesh = plsc.VectorSubcoreMesh(core_axis_name="c", subcore_axis_name="s")


@functools.partial(
    pl.kernel,
    mesh=_mesh,
    out_type=jax.ShapeDtypeStruct((TOTAL, DIM), jnp.float32),
    scratch_types=[
        pltpu.VMEM((NCHUNK, CHUNK), jnp.int32),
        pltpu.VMEM((2, CHUNK, DIM), jnp.float32),
        pltpu.VMEM_SHARED((CHARSET, DIM), jnp.float32),
        pltpu.SemaphoreType.DMA,
    ],
)
def _sc_gather(table_hbm, idx_hbm, out_hbm, idx_v, rows_v, tab_sh, sem):
    sid = lax.axis_index("s")
    wid = sid * NC + lax.axis_index("c")

    @pl.when(sid == 0)
    def _():
        pltpu.sync_copy(table_hbm, tab_sh)

    pltpu.sync_copy(idx_hbm.at[wid], idx_v)
    plsc.subcore_barrier()
    base0 = wid * PER_W

    pltpu.async_copy(tab_sh.at[idx_v.at[0]], rows_v.at[0], sem)

    def body(j, carry):
        buf = lax.rem(j, 2)
        pltpu.make_async_copy(tab_sh.at[idx_v.at[j]], rows_v.at[buf],
                              sem).wait()

        @pl.when(j + 1 < NCHUNK)
        def _():
            pltpu.async_copy(tab_sh.at[idx_v.at[j + 1]],
                             rows_v.at[1 - buf], sem)

        pltpu.sync_copy(rows_v.at[buf],
                        out_hbm.at[pl.ds(base0 + j * CHUNK, CHUNK)])
        return carry

    lax.fori_loop(0, NCHUNK, body, 0)


def kernel(char_seq, mapping_weight, char_emb_weight):
    table = _compute_table(mapping_weight, char_emb_weight)
    idx = char_seq.reshape(NW, NCHUNK, CHUNK).astype(jnp.int32)
    out = _sc_gather(table, idx)
    return out.reshape(BATCH, SEQ, DIM)


# speedup vs baseline: 17.0468x; 1.0372x over previous
import functools

import jax
import jax.numpy as jnp
from jax import lax
from jax.experimental import pallas as pl
from jax.experimental.pallas import tpu as pltpu
from jax.experimental.pallas import tpu_sc as plsc

CHARSET = 1000
UNIVERSAL = 1024
DIM = 128
BATCH = 4096
SEQ = 200

NC = 2
NS = 16
NW = NC * NS

TOTAL = BATCH * SEQ
PER_W = TOTAL // NW
CHUNK = 128
NCHUNK = PER_W // CHUNK
GROUP = 2
GROUP_ROWS = GROUP * CHUNK
NGROUP = NCHUNK // GROUP


def _matmul_body(a_ref, b_ref, o_ref):
    o_ref[...] = jnp.dot(a_ref[...], b_ref[...],
                         preferred_element_type=jnp.float32)


def _compute_table(mapping_weight, char_emb_weight):
    return pl.pallas_call(
        _matmul_body,
        out_shape=jax.ShapeDtypeStruct((CHARSET, DIM), jnp.float32),
    )(mapping_weight, char_emb_weight)


_mesh = plsc.VectorSubcoreMesh(core_axis_name="c", subcore_axis_name="s")


@functools.partial(
    pl.kernel,
    mesh=_mesh,
    out_type=jax.ShapeDtypeStruct((NW * NGROUP, GROUP_ROWS, DIM),
                                  jnp.float32),
    scratch_types=[
        pltpu.VMEM((NCHUNK, CHUNK), jnp.int32),
        pltpu.VMEM((2, GROUP_ROWS, DIM), jnp.float32),
        pltpu.VMEM_SHARED((CHARSET, DIM), jnp.float32),
        pltpu.SemaphoreType.DMA,
        pltpu.SemaphoreType.DMA,
    ],
)
def _sc_gather(table_hbm, idx_hbm, out_hbm, idx_v, rows_v, tab_sh,
               gsem, ssem):
    sid = lax.axis_index("s")
    wid = sid * NC + lax.axis_index("c")

    @pl.when(sid == 0)
    def _():
        pltpu.sync_copy(table_hbm, tab_sh)

    pltpu.sync_copy(idx_hbm.at[wid], idx_v)
    plsc.subcore_barrier()
    gbase = wid * NGROUP

    def start_gathers(g, buf):
        for k in range(GROUP):
            pltpu.async_copy(tab_sh.at[idx_v.at[GROUP * g + k]],
                             rows_v.at[buf, pl.ds(k * CHUNK, CHUNK)], gsem)

    def wait_gathers(g, buf):
        for k in range(GROUP):
            pltpu.make_async_copy(
                tab_sh.at[idx_v.at[GROUP * g + k]],
                rows_v.at[buf, pl.ds(k * CHUNK, CHUNK)], gsem).wait()

    def scatter_copy(g, buf):
        return pltpu.make_async_copy(rows_v.at[buf], out_hbm.at[gbase + g],
                                     ssem)

    start_gathers(0, 0)

    def body(g, carry):
        buf = lax.rem(g, 2)
        wait_gathers(g, buf)
        scatter_copy(g, buf).start()

        @pl.when(g + 1 < NGROUP)
        def _():
            @pl.when(g >= 1)
            def _():
                scatter_copy(g - 1, 1 - buf).wait()

            start_gathers(g + 1, 1 - buf)

        return carry

    lax.fori_loop(0, NGROUP, body, 0)
    scatter_copy(NGROUP - 2, lax.rem(NGROUP - 2, 2)).wait()
    scatter_copy(NGROUP - 1, lax.rem(NGROUP - 1, 2)).wait()


def kernel(char_seq, mapping_weight, char_emb_weight):
    table = _compute_table(mapping_weight, char_emb_weight)
    idx = char_seq.reshape(NW, NCHUNK, CHUNK).astype(jnp.int32)
    out = _sc_gather(table, idx)
    return out.reshape(BATCH, SEQ, DIM)


# speedup vs baseline: 17.7541x; 1.0415x over previous
import functools

import jax
import jax.numpy as jnp
from jax import lax
from jax.experimental import pallas as pl
from jax.experimental.pallas import tpu as pltpu
from jax.experimental.pallas import tpu_sc as plsc

CHARSET = 1000
UNIVERSAL = 1024
DIM = 128
BATCH = 4096
SEQ = 200

NC = 2
NS = 16
NW = NC * NS

TOTAL = BATCH * SEQ
PER_W = TOTAL // NW
CHUNK = 128
NCHUNK = PER_W // CHUNK
GROUP = 1
GROUP_ROWS = GROUP * CHUNK
NGROUP = NCHUNK // GROUP
NBUF = 4


def _matmul_body(a_ref, b_ref, o_ref):
    o_ref[...] = jnp.dot(a_ref[...], b_ref[...],
                         preferred_element_type=jnp.float32)


def _compute_table(mapping_weight, char_emb_weight):
    return pl.pallas_call(
        _matmul_body,
        out_shape=jax.ShapeDtypeStruct((CHARSET, DIM), jnp.float32),
    )(mapping_weight, char_emb_weight)


_mesh = plsc.VectorSubcoreMesh(core_axis_name="c", subcore_axis_name="s")


@functools.partial(
    pl.kernel,
    mesh=_mesh,
    out_type=jax.ShapeDtypeStruct((NW * NGROUP, GROUP_ROWS, DIM),
                                  jnp.float32),
    scratch_types=[
        pltpu.VMEM((NCHUNK, CHUNK), jnp.int32),
        pltpu.VMEM((NBUF, GROUP_ROWS, DIM), jnp.float32),
        pltpu.VMEM_SHARED((CHARSET, DIM), jnp.float32),
        pltpu.SemaphoreType.DMA,
        pltpu.SemaphoreType.DMA,
    ],
)
def _sc_gather(table_hbm, idx_hbm, out_hbm, idx_v, rows_v, tab_sh,
               gsem, ssem):
    sid = lax.axis_index("s")
    wid = sid * NC + lax.axis_index("c")

    @pl.when(sid == 0)
    def _():
        pltpu.sync_copy(table_hbm, tab_sh)

    pltpu.sync_copy(idx_hbm.at[wid], idx_v)
    plsc.subcore_barrier()
    gbase = wid * NGROUP

    def start_gathers(g, buf):
        for k in range(GROUP):
            pltpu.async_copy(tab_sh.at[idx_v.at[GROUP * g + k]],
                             rows_v.at[buf, pl.ds(k * CHUNK, CHUNK)], gsem)

    def wait_gathers(g, buf):
        for k in range(GROUP):
            pltpu.make_async_copy(
                tab_sh.at[idx_v.at[GROUP * g + k]],
                rows_v.at[buf, pl.ds(k * CHUNK, CHUNK)], gsem).wait()

    def scatter_copy(g, buf):
        return pltpu.make_async_copy(rows_v.at[buf], out_hbm.at[gbase + g],
                                     ssem)

    start_gathers(0, 0)
    start_gathers(1, 1)

    def body(g, carry):
        buf = lax.rem(g, NBUF)
        wait_gathers(g, buf)
        scatter_copy(g, buf).start()

        @pl.when(g + 2 < NGROUP)
        def _():
            @pl.when(g >= NBUF - 2)
            def _():
                scatter_copy(g - (NBUF - 2), lax.rem(g + 2, NBUF)).wait()

            start_gathers(g + 2, lax.rem(g + 2, NBUF))

        return carry

    lax.fori_loop(0, NGROUP, body, 0)
    for t in range(NGROUP - NBUF, NGROUP):
        scatter_copy(t, lax.rem(t, NBUF)).wait()


def kernel(char_seq, mapping_weight, char_emb_weight):
    table = _compute_table(mapping_weight, char_emb_weight)
    idx = char_seq.reshape(NW, NCHUNK, CHUNK).astype(jnp.int32)
    out = _sc_gather(table, idx)
    return out.reshape(BATCH, SEQ, DIM)


# speedup vs baseline: 17.7741x; 1.0011x over previous
import functools

import jax
import jax.numpy as jnp
from jax import lax
from jax.experimental import pallas as pl
from jax.experimental.pallas import tpu as pltpu
from jax.experimental.pallas import tpu_sc as plsc

CHARSET = 1000
UNIVERSAL = 1024
DIM = 128
BATCH = 4096
SEQ = 200

NC = 2
NS = 16
NW = NC * NS

TOTAL = BATCH * SEQ
PER_W = TOTAL // NW
CHUNK = 128
NCHUNK = PER_W // CHUNK
GROUP = 1
GROUP_ROWS = GROUP * CHUNK
NGROUP = NCHUNK // GROUP
NBUF = 5


def _matmul_body(a_ref, b_ref, o_ref):
    o_ref[...] = jnp.dot(a_ref[...], b_ref[...],
                         preferred_element_type=jnp.float32)


def _compute_table(mapping_weight, char_emb_weight):
    return pl.pallas_call(
        _matmul_body,
        out_shape=jax.ShapeDtypeStruct((CHARSET, DIM), jnp.float32),
    )(mapping_weight, char_emb_weight)


_mesh = plsc.VectorSubcoreMesh(core_axis_name="c", subcore_axis_name="s")


@functools.partial(
    pl.kernel,
    mesh=_mesh,
    out_type=jax.ShapeDtypeStruct((NW * NGROUP, GROUP_ROWS, DIM),
                                  jnp.float32),
    scratch_types=[
        pltpu.VMEM((NCHUNK, CHUNK), jnp.int32),
        pltpu.VMEM((NBUF, GROUP_ROWS, DIM), jnp.float32),
        pltpu.VMEM_SHARED((CHARSET, DIM), jnp.float32),
        pltpu.SemaphoreType.DMA,
        pltpu.SemaphoreType.DMA,
    ],
)
def _sc_gather(table_hbm, idx_hbm, out_hbm, idx_v, rows_v, tab_sh,
               gsem, ssem):
    sid = lax.axis_index("s")
    wid = sid * NC + lax.axis_index("c")

    @pl.when(sid == 0)
    def _():
        pltpu.sync_copy(table_hbm, tab_sh)

    pltpu.sync_copy(idx_hbm.at[wid], idx_v)
    plsc.subcore_barrier()
    gbase = wid * NGROUP

    def start_gathers(g, buf):
        for k in range(GROUP):
            pltpu.async_copy(tab_sh.at[idx_v.at[GROUP * g + k]],
                             rows_v.at[buf, pl.ds(k * CHUNK, CHUNK)], gsem)

    def wait_gathers(g, buf):
        for k in range(GROUP):
            pltpu.make_async_copy(
                tab_sh.at[idx_v.at[GROUP * g + k]],
                rows_v.at[buf, pl.ds(k * CHUNK, CHUNK)], gsem).wait()

    def scatter_copy(g, buf):
        return pltpu.make_async_copy(rows_v.at[buf], out_hbm.at[gbase + g],
                                     ssem)

    start_gathers(0, 0)
    start_gathers(1, 1)

    def body(g, carry):
        buf = lax.rem(g, NBUF)
        wait_gathers(g, buf)
        scatter_copy(g, buf).start()

        @pl.when(g + 2 < NGROUP)
        def _():
            @pl.when(g >= NBUF - 2)
            def _():
                scatter_copy(g - (NBUF - 2), lax.rem(g + 2, NBUF)).wait()

            start_gathers(g + 2, lax.rem(g + 2, NBUF))

        return carry

    lax.fori_loop(0, NGROUP, body, 0)
    for t in range(NGROUP - NBUF, NGROUP):
        scatter_copy(t, lax.rem(t, NBUF)).wait()


def kernel(char_seq, mapping_weight, char_emb_weight):
    table = _compute_table(mapping_weight, char_emb_weight)
    idx = char_seq.reshape(NW, NCHUNK, CHUNK).astype(jnp.int32)
    out = _sc_gather(table, idx)
    return out.reshape(BATCH, SEQ, DIM)
